# baseline (speedup 1.0000x reference)
import functools

import jax
import jax.numpy as jnp
from jax import lax
from jax.experimental import pallas as pl
from jax.experimental.pallas import tpu as pltpu
from jax.experimental.pallas import tpu_sc as plsc


def _raw_indices():
    key = jax.random.key(42)
    gap = jax.random.randint(key, (4,), 4, 16, dtype=jnp.int32)
    gap = jnp.sort(gap)
    return jnp.concatenate((jnp.zeros((1,), dtype=gap.dtype), gap))


def _temporal_indices():
    try:
        with jax.ensure_compile_time_eval():
            return tuple(int(v) for v in _raw_indices())
    except Exception:
        idx = _raw_indices()
        return tuple(idx[j] for j in range(idx.shape[0]))


def _ring_copy(slabs, n, bufs, isems, osems, nbuf, depth):
    ins = [None] * n
    outs = [None] * n

    def start_in(i):
        src, _ = slabs(i)
        ins[i] = pltpu.async_copy(src, bufs.at[i % nbuf], isems[i % nbuf])

    for i in range(min(depth, n)):
        start_in(i)
    for i in range(n):
        nxt = i + depth
        if nxt < n:
            if nxt >= nbuf:
                outs[nxt - nbuf].wait()
            start_in(nxt)
        ins[i].wait()
        _, dst = slabs(i)
        outs[i] = pltpu.async_copy(bufs.at[i % nbuf], dst, osems[i % nbuf])
    for i in range(max(0, n - nbuf), n):
        outs[i].wait()


def _sc_gather(x3, idx):
    GT, H, W = x3.shape
    T = 32
    G = GT // T
    OT = 5
    R = G * OT
    NQ = 4
    HQ = H // NQ
    NCHUNK = R * NQ

    info = plsc.get_sparse_core_info()
    NC, NS = info.num_cores, info.num_subcores
    NW = NC * NS
    PER_W = NCHUNK // NW

    lut = idx[0]
    for j in range(1, OT):
        lut = lut + (idx[j] << (5 * j))

    mesh = plsc.VectorSubcoreMesh(core_axis_name="c", subcore_axis_name="s")
    NBUF = 8
    DEPTH = 4

    @functools.partial(
        pl.kernel,
        out_type=jax.ShapeDtypeStruct((R, H, W), jnp.float32),
        mesh=mesh,
        scratch_types=(
            [pltpu.VMEM((NBUF, HQ, W), jnp.float32)]
            + [pltpu.SemaphoreType.DMA] * (2 * NBUF)
        ),
        compiler_params=pltpu.CompilerParams(
            disable_bounds_checks=True,
            disable_semaphore_checks=True,
            skip_device_barrier=True,
            use_tc_tiling_on_sc=True,
        ),
    )
    def body(x_hbm, out_hbm, bufs, *sems):
        w = lax.axis_index("s") * NC + lax.axis_index("c")

        def slabs(i):
            k = w * PER_W + i
            r = k >> 2
            q = k & 3
            g = (r * 52429) >> 18
            t = r - 5 * g
            st = (lut >> (5 * t)) & 31
            src_row = g * T + st
            return (
                x_hbm.at[src_row, pl.ds(q * HQ, HQ), :],
                out_hbm.at[r, pl.ds(q * HQ, HQ), :],
            )

        _ring_copy(slabs, PER_W, bufs, sems[:NBUF], sems[NBUF:], NBUF, DEPTH)

    return body(x3)


def kernel(x):
    B, C, T, H, W = x.shape
    idx = _temporal_indices()
    OT = len(idx)
    x3 = x.reshape(B * C * T, H, W)
    out3 = _sc_gather(x3, idx)
    return out3.reshape(B, C, OT, H, W)


# speedup vs baseline: 1.0042x; 1.0042x over previous
import functools

import jax
import jax.numpy as jnp
from jax import lax
from jax.experimental import pallas as pl
from jax.experimental.pallas import tpu as pltpu
from jax.experimental.pallas import tpu_sc as plsc


def _raw_indices():
    key = jax.random.key(42)
    gap = jax.random.randint(key, (4,), 4, 16, dtype=jnp.int32)
    gap = jnp.sort(gap)
    return jnp.concatenate((jnp.zeros((1,), dtype=gap.dtype), gap))


def _temporal_indices():
    try:
        with jax.ensure_compile_time_eval():
            return tuple(int(v) for v in _raw_indices())
    except Exception:
        idx = _raw_indices()
        return tuple(idx[j] for j in range(idx.shape[0]))


def _ring_copy(slabs, n, bufs, isems, osems, nbuf, depth):
    ins = [None] * n
    outs = [None] * n

    def start_in(i):
        src, _ = slabs(i)
        ins[i] = pltpu.async_copy(src, bufs.at[i % nbuf], isems[i % nbuf])

    for i in range(min(depth, n)):
        start_in(i)
    for i in range(n):
        nxt = i + depth
        if nxt < n:
            if nxt >= nbuf:
                outs[nxt - nbuf].wait()
            start_in(nxt)
        ins[i].wait()
        _, dst = slabs(i)
        outs[i] = pltpu.async_copy(bufs.at[i % nbuf], dst, osems[i % nbuf])
    for i in range(max(0, n - nbuf), n):
        outs[i].wait()


def _sc_gather(x3, idx):
    GT, H, W = x3.shape
    T = 32
    G = GT // T
    OT = 5
    R = G * OT
    NQ = 4
    HQ = H // NQ
    NCHUNK = R * NQ

    info = plsc.get_sparse_core_info()
    NC, NS = info.num_cores, info.num_subcores
    NW = NC * NS
    PER_W = NCHUNK // NW

    lut = idx[0]
    for j in range(1, OT):
        lut = lut + (idx[j] << (5 * j))

    mesh = plsc.VectorSubcoreMesh(core_axis_name="c", subcore_axis_name="s")
    NBUF = 8
    DEPTH = 4

    @functools.partial(
        pl.kernel,
        out_type=jax.ShapeDtypeStruct((R, H, W), jnp.float32),
        mesh=mesh,
        scratch_types=(
            [pltpu.VMEM((NBUF, HQ, W), jnp.float32)]
            + [pltpu.SemaphoreType.DMA] * (2 * NBUF)
        ),
        compiler_params=pltpu.CompilerParams(
            disable_bounds_checks=True,
            disable_semaphore_checks=True,
            skip_device_barrier=True,
        ),
    )
    def body(x_hbm, out_hbm, bufs, *sems):
        w = lax.axis_index("s") * NC + lax.axis_index("c")

        def slabs(i):
            k = w * PER_W + i
            r = k >> 2
            q = k & 3
            g = (r * 52429) >> 18
            t = r - 5 * g
            st = (lut >> (5 * t)) & 31
            src_row = g * T + st
            return (
                x_hbm.at[src_row, pl.ds(q * HQ, HQ), :],
                out_hbm.at[r, pl.ds(q * HQ, HQ), :],
            )

        _ring_copy(slabs, PER_W, bufs, sems[:NBUF], sems[NBUF:], NBUF, DEPTH)

    return body(x3)


def kernel(x):
    B, C, T, H, W = x.shape
    idx = _temporal_indices()
    OT = len(idx)
    x3 = x.reshape(B * C * T, H, W)
    out3 = _sc_gather(x3, idx)
    return out3.reshape(B, C, OT, H, W)
